# initial kernel scaffold (unmeasured)
import jax
import jax.numpy as jnp
from jax import lax
from jax.experimental import pallas as pl
from jax.experimental.pallas import tpu as pltpu

N_DEV = 4


def kernel(x, W1, W2):
    m_per, d = x.shape
    f_per = W1.shape[1]

    def body(x_ref, w1_ref, w2_ref, out_ref,
             xfull, pacc, rsbuf,
             ag_send, ag_recv, rs_send, rs_recv):
        my = lax.axis_index("i")
        left = lax.rem(my + N_DEV - 1, N_DEV)
        right = lax.rem(my + 1, N_DEV)

        bar = pltpu.get_barrier_semaphore()
        for nbr in (left, right):
            pl.semaphore_signal(bar, inc=1, device_id=(nbr,),
                                device_id_type=pl.DeviceIdType.MESH)
        pl.semaphore_wait(bar, 2)

        xfull[pl.ds(my * m_per, m_per), :] = x_ref[...].astype(jnp.bfloat16)

        for h in range(N_DEV - 1):
            org = lax.rem(my - h + N_DEV, N_DEV)
            rdma = pltpu.make_async_remote_copy(
                src_ref=xfull.at[pl.ds(org * m_per, m_per), :],
                dst_ref=xfull.at[pl.ds(org * m_per, m_per), :],
                send_sem=ag_send.at[h],
                recv_sem=ag_recv.at[h],
                device_id=(right,),
                device_id_type=pl.DeviceIdType.MESH,
            )
            rdma.start()
            rdma.wait()

        w1 = w1_ref[...].astype(jnp.bfloat16)
        w2 = w2_ref[...].astype(jnp.bfloat16)
        for blk in range(N_DEV):
            xb = xfull[pl.ds(blk * m_per, m_per), :]
            hb = jnp.dot(xb, w1, preferred_element_type=jnp.float32)
            hb = hb * jax.nn.sigmoid(hb)
            pb = jnp.dot(hb.astype(jnp.bfloat16), w2,
                         preferred_element_type=jnp.float32)
            pacc[pl.ds(blk * m_per, m_per), :] = pb

        for s in range(N_DEV - 1):
            send_c = lax.rem(my + s + 1, N_DEV)
            recv_c = lax.rem(my + s + 2, N_DEV)
            rdma = pltpu.make_async_remote_copy(
                src_ref=pacc.at[pl.ds(send_c * m_per, m_per), :],
                dst_ref=rsbuf.at[s],
                send_sem=rs_send.at[s],
                recv_sem=rs_recv.at[s],
                device_id=(left,),
                device_id_type=pl.DeviceIdType.MESH,
            )
            rdma.start()
            rdma.wait()
            pacc[pl.ds(recv_c * m_per, m_per), :] = (
                pacc[pl.ds(recv_c * m_per, m_per), :] + rsbuf[s]
            )

        out_ref[...] = pacc[pl.ds(my * m_per, m_per), :]

    return pl.pallas_call(
        body,
        out_shape=jax.ShapeDtypeStruct((m_per, d), jnp.float32),
        in_specs=[pl.BlockSpec(memory_space=pltpu.VMEM)] * 3,
        out_specs=pl.BlockSpec(memory_space=pltpu.VMEM),
        scratch_shapes=[
            pltpu.VMEM((N_DEV * m_per, d), jnp.bfloat16),
            pltpu.VMEM((N_DEV * m_per, d), jnp.float32),
            pltpu.VMEM((N_DEV - 1, m_per, d), jnp.float32),
            pltpu.SemaphoreType.DMA((N_DEV - 1,)),
            pltpu.SemaphoreType.DMA((N_DEV - 1,)),
            pltpu.SemaphoreType.DMA((N_DEV - 1,)),
            pltpu.SemaphoreType.DMA((N_DEV - 1,)),
        ],
        compiler_params=pltpu.CompilerParams(collective_id=0),
    )(x, W1, W2)


# baseline (device time: 262292 ns/iter reference)
import jax
import jax.numpy as jnp
from jax import lax
from jax.experimental import pallas as pl
from jax.experimental.pallas import tpu as pltpu

N_DEV = 4
F_CHUNK = 512


def kernel(x, W1, W2):
    m_per, d = x.shape
    f_per = W1.shape[1]

    def body(x_ref, w1_ref, w2_ref, out_ref,
             xfull, pacc, rsbuf,
             ag_send, ag_recv, rs_send, rs_recv):
        my = lax.axis_index("i")
        left = lax.rem(my + N_DEV - 1, N_DEV)
        right = lax.rem(my + 1, N_DEV)

        bar = pltpu.get_barrier_semaphore()
        for nbr in (left, right):
            pl.semaphore_signal(bar, inc=1, device_id=(nbr,),
                                device_id_type=pl.DeviceIdType.MESH)
        pl.semaphore_wait(bar, 2)

        xfull[pl.ds(my * m_per, m_per), :] = x_ref[...]

        for h in range(N_DEV - 1):
            org = lax.rem(my - h + N_DEV, N_DEV)
            rdma = pltpu.make_async_remote_copy(
                src_ref=xfull.at[pl.ds(org * m_per, m_per), :],
                dst_ref=xfull.at[pl.ds(org * m_per, m_per), :],
                send_sem=ag_send.at[h],
                recv_sem=ag_recv.at[h],
                device_id=(right,),
                device_id_type=pl.DeviceIdType.MESH,
            )
            rdma.start()
            rdma.wait()

        for blk in range(N_DEV):
            xb = xfull[pl.ds(blk * m_per, m_per), :]
            p = jnp.zeros((m_per, d), jnp.float32)
            for fc in range(f_per // F_CHUNK):
                w1c = w1_ref[:, pl.ds(fc * F_CHUNK, F_CHUNK)]
                hc = jnp.dot(xb, w1c, preferred_element_type=jnp.float32)
                hc = hc * jax.nn.sigmoid(hc)
                w2c = w2_ref[pl.ds(fc * F_CHUNK, F_CHUNK), :]
                p = p + jnp.dot(hc.astype(jnp.bfloat16), w2c,
                                preferred_element_type=jnp.float32)
            pacc[pl.ds(blk * m_per, m_per), :] = p.astype(jnp.bfloat16)

        for s in range(N_DEV - 1):
            send_c = lax.rem(my + s + 1, N_DEV)
            recv_c = lax.rem(my + s + 2, N_DEV)
            rdma = pltpu.make_async_remote_copy(
                src_ref=pacc.at[pl.ds(send_c * m_per, m_per), :],
                dst_ref=rsbuf.at[s],
                send_sem=rs_send.at[s],
                recv_sem=rs_recv.at[s],
                device_id=(left,),
                device_id_type=pl.DeviceIdType.MESH,
            )
            rdma.start()
            rdma.wait()
            pacc[pl.ds(recv_c * m_per, m_per), :] = (
                pacc[pl.ds(recv_c * m_per, m_per), :] + rsbuf[s]
            )

        out_ref[...] = pacc[pl.ds(my * m_per, m_per), :].astype(jnp.float32)

    f = pl.pallas_call(
        body,
        out_shape=jax.ShapeDtypeStruct((m_per, d), jnp.float32),
        in_specs=[pl.BlockSpec(memory_space=pltpu.VMEM)] * 3,
        out_specs=pl.BlockSpec(memory_space=pltpu.VMEM),
        scratch_shapes=[
            pltpu.VMEM((N_DEV * m_per, d), jnp.bfloat16),
            pltpu.VMEM((N_DEV * m_per, d), jnp.bfloat16),
            pltpu.VMEM((N_DEV - 1, m_per, d), jnp.bfloat16),
            pltpu.SemaphoreType.DMA((N_DEV - 1,)),
            pltpu.SemaphoreType.DMA((N_DEV - 1,)),
            pltpu.SemaphoreType.DMA((N_DEV - 1,)),
            pltpu.SemaphoreType.DMA((N_DEV - 1,)),
        ],
        compiler_params=pltpu.CompilerParams(
            collective_id=0, vmem_limit_bytes=60 * 1024 * 1024),
    )
    return f(x.astype(jnp.bfloat16), W1.astype(jnp.bfloat16),
             W2.astype(jnp.bfloat16))


# device time: 154825 ns/iter; 1.6941x vs baseline; 1.6941x over previous
import jax
import jax.numpy as jnp
from jax import lax
from jax.experimental import pallas as pl
from jax.experimental.pallas import tpu as pltpu

N_DEV = 4
F_CHUNK = 512


def kernel(x, W1, W2):
    m_per, d = x.shape
    f_per = W1.shape[1]

    def compute_partial(xb, w1_ref, w2_ref):
        p = jnp.zeros((m_per, d), jnp.float32)
        for fc in range(f_per // F_CHUNK):
            w1c = w1_ref[:, pl.ds(fc * F_CHUNK, F_CHUNK)]
            hc = jnp.dot(xb, w1c, preferred_element_type=jnp.float32)
            hc = hc * jax.nn.sigmoid(hc)
            w2c = w2_ref[pl.ds(fc * F_CHUNK, F_CHUNK), :]
            p = p + jnp.dot(hc.astype(jnp.bfloat16), w2c,
                            preferred_element_type=jnp.float32)
        return p

    def body(x_ref, w1_ref, w2_ref, out_ref,
             xfull, pown, psend, rsbuf,
             ag_send, ag_recv, rs_send, rs_recv):
        my = lax.axis_index("i")
        left = lax.rem(my + N_DEV - 1, N_DEV)
        right = lax.rem(my + 1, N_DEV)

        bar = pltpu.get_barrier_semaphore()
        for nbr in (left, right):
            pl.semaphore_signal(bar, inc=1, device_id=(nbr,),
                                device_id_type=pl.DeviceIdType.MESH)
        pl.semaphore_wait(bar, 2)

        xfull[pl.ds(my * m_per, m_per), :] = x_ref[...]

        rs_target = [left, lax.rem(my + 2, N_DEV), right]
        rs_slot = [2, 1, 0]

        ag_rdmas = []
        rs_rdmas = []
        for k in range(N_DEV):
            if k > 0:
                ag_rdmas[k - 1].wait_recv()
            org = lax.rem(my - k + N_DEV, N_DEV)
            if k < N_DEV - 1:
                rdma = pltpu.make_async_remote_copy(
                    src_ref=xfull.at[pl.ds(org * m_per, m_per), :],
                    dst_ref=xfull.at[pl.ds(org * m_per, m_per), :],
                    send_sem=ag_send.at[k],
                    recv_sem=ag_recv.at[k],
                    device_id=(right,),
                    device_id_type=pl.DeviceIdType.MESH,
                )
                rdma.start()
                ag_rdmas.append(rdma)

            p = compute_partial(xfull[pl.ds(org * m_per, m_per), :],
                                w1_ref, w2_ref)
            if k == 0:
                pown[...] = p
            else:
                psend[k - 1] = p.astype(jnp.bfloat16)
                rdma = pltpu.make_async_remote_copy(
                    src_ref=psend.at[k - 1],
                    dst_ref=rsbuf.at[rs_slot[k - 1]],
                    send_sem=rs_send.at[k - 1],
                    recv_sem=rs_recv.at[rs_slot[k - 1]],
                    device_id=(rs_target[k - 1],),
                    device_id_type=pl.DeviceIdType.MESH,
                )
                rdma.start()
                rs_rdmas.append(rdma)

        for j in range(N_DEV - 1):
            recv = pltpu.make_async_remote_copy(
                src_ref=rsbuf.at[j],
                dst_ref=rsbuf.at[j],
                send_sem=rs_send.at[j],
                recv_sem=rs_recv.at[j],
                device_id=(my,),
                device_id_type=pl.DeviceIdType.MESH,
            )
            recv.wait_recv()

        out_ref[...] = (pown[...]
                        + rsbuf[0].astype(jnp.float32)
                        + rsbuf[1].astype(jnp.float32)
                        + rsbuf[2].astype(jnp.float32))

        for rdma in ag_rdmas + rs_rdmas:
            rdma.wait_send()

    f = pl.pallas_call(
        body,
        out_shape=jax.ShapeDtypeStruct((m_per, d), jnp.float32),
        in_specs=[pl.BlockSpec(memory_space=pltpu.VMEM)] * 3,
        out_specs=pl.BlockSpec(memory_space=pltpu.VMEM),
        scratch_shapes=[
            pltpu.VMEM((N_DEV * m_per, d), jnp.bfloat16),
            pltpu.VMEM((m_per, d), jnp.float32),
            pltpu.VMEM((N_DEV - 1, m_per, d), jnp.bfloat16),
            pltpu.VMEM((N_DEV - 1, m_per, d), jnp.bfloat16),
            pltpu.SemaphoreType.DMA((N_DEV - 1,)),
            pltpu.SemaphoreType.DMA((N_DEV - 1,)),
            pltpu.SemaphoreType.DMA((N_DEV - 1,)),
            pltpu.SemaphoreType.DMA((N_DEV - 1,)),
        ],
        compiler_params=pltpu.CompilerParams(
            collective_id=0, vmem_limit_bytes=60 * 1024 * 1024),
    )
    return f(x.astype(jnp.bfloat16), W1.astype(jnp.bfloat16),
             W2.astype(jnp.bfloat16))


# device time: 146264 ns/iter; 1.7933x vs baseline; 1.0585x over previous
import jax
import jax.numpy as jnp
from jax import lax
from jax.experimental import pallas as pl
from jax.experimental.pallas import tpu as pltpu

N_DEV = 4
F_CHUNK = 512
N_HALF = 2


def kernel(x, W1, W2):
    m_per, d = x.shape
    f_per = W1.shape[1]
    m_half = m_per // N_HALF

    def compute_partial(xb, w1_ref, w2_ref):
        p = jnp.zeros((m_half, d), jnp.float32)
        for fc in range(f_per // F_CHUNK):
            w1c = w1_ref[:, pl.ds(fc * F_CHUNK, F_CHUNK)]
            hc = jnp.dot(xb, w1c, preferred_element_type=jnp.float32)
            hc = hc * jax.nn.sigmoid(hc)
            w2c = w2_ref[pl.ds(fc * F_CHUNK, F_CHUNK), :]
            p = p + jnp.dot(hc.astype(jnp.bfloat16), w2c,
                            preferred_element_type=jnp.float32)
        return p

    def body(x_ref, w1_ref, w2_ref, out_ref,
             xfull, pown, pk1, pk2, pk3, fleg, sumb, dirb,
             ag_send, ag_recv,
             k1_send, k2_send, k3_send,
             fleg_recv, sum_recv, dir_recv):
        my = lax.axis_index("i")
        left = lax.rem(my + N_DEV - 1, N_DEV)
        right = lax.rem(my + 1, N_DEV)

        bar = pltpu.get_barrier_semaphore()
        for nbr in (left, right):
            pl.semaphore_signal(bar, inc=1, device_id=(nbr,),
                                device_id_type=pl.DeviceIdType.MESH)
        pl.semaphore_wait(bar, 2)

        def xslice(org, h):
            return pl.ds(org * m_per + h * m_half, m_half)

        def hslice(h):
            return pl.ds(h * m_half, m_half)

        waits = []

        xfull[pl.ds(my * m_per, m_per), :] = x_ref[...]

        ag_rdmas = {}
        org0 = my
        for h in range(N_HALF):
            rdma = pltpu.make_async_remote_copy(
                src_ref=xfull.at[xslice(org0, h), :],
                dst_ref=xfull.at[xslice(org0, h), :],
                send_sem=ag_send.at[h],
                recv_sem=ag_recv.at[h],
                device_id=(right,),
                device_id_type=pl.DeviceIdType.MESH,
            )
            rdma.start()
            ag_rdmas[(0, h)] = rdma
            waits.append(rdma)
        for h in range(N_HALF):
            pown[hslice(h), :] = compute_partial(
                xfull[xslice(org0, h), :], w1_ref, w2_ref)

        def ag_step(k):
            org = lax.rem(my - k + N_DEV, N_DEV)
            outs = []
            for h in range(N_HALF):
                ag_rdmas[(k - 1, h)].wait_recv()
                if k < N_DEV - 1:
                    rdma = pltpu.make_async_remote_copy(
                        src_ref=xfull.at[xslice(org, h), :],
                        dst_ref=xfull.at[xslice(org, h), :],
                        send_sem=ag_send.at[2 * k + h],
                        recv_sem=ag_recv.at[2 * k + h],
                        device_id=(right,),
                        device_id_type=pl.DeviceIdType.MESH,
                    )
                    rdma.start()
                    ag_rdmas[(k, h)] = rdma
                    waits.append(rdma)
                outs.append(compute_partial(
                    xfull[xslice(org, h), :], w1_ref, w2_ref))
            return outs

        def send(buf, h, sem, target, dst_buf, dst_sem):
            rdma = pltpu.make_async_remote_copy(
                src_ref=buf.at[hslice(h), :],
                dst_ref=dst_buf.at[hslice(h), :],
                send_sem=sem.at[h],
                recv_sem=dst_sem.at[h],
                device_id=(target,),
                device_id_type=pl.DeviceIdType.MESH,
            )
            rdma.start()
            waits.append(rdma)

        def wait_recv(buf, sem, h):
            rdma = pltpu.make_async_remote_copy(
                src_ref=buf.at[hslice(h), :],
                dst_ref=buf.at[hslice(h), :],
                send_sem=sem.at[h],
                recv_sem=sem.at[h],
                device_id=(my,),
                device_id_type=pl.DeviceIdType.MESH,
            )
            rdma.wait_recv()

        for h, p in enumerate(ag_step(1)):
            pk1[hslice(h), :] = p.astype(jnp.bfloat16)

        for h, p in enumerate(ag_step(2)):
            pk2[hslice(h), :] = p.astype(jnp.bfloat16)
            send(pk2, h, k2_send, left, fleg, fleg_recv)

        org3 = lax.rem(my + 1, N_DEV)
        for h in range(N_HALF):
            ag_rdmas[(2, h)].wait_recv()
            pk3[hslice(h), :] = compute_partial(
                xfull[xslice(org3, h), :], w1_ref, w2_ref).astype(jnp.bfloat16)
            send(pk3, h, k3_send, right, dirb, dir_recv)
            wait_recv(fleg, fleg_recv, h)
            pk1[hslice(h), :] = pk1[hslice(h), :] + fleg[hslice(h), :]
            send(pk1, h, k1_send, left, sumb, sum_recv)

        for h in range(N_HALF):
            wait_recv(sumb, sum_recv, h)
            wait_recv(dirb, dir_recv, h)
            out_ref[hslice(h), :] = (
                pown[hslice(h), :]
                + sumb[hslice(h), :].astype(jnp.float32)
                + dirb[hslice(h), :].astype(jnp.float32))

        for rdma in waits:
            rdma.wait_send()

    f = pl.pallas_call(
        body,
        out_shape=jax.ShapeDtypeStruct((m_per, d), jnp.float32),
        in_specs=[pl.BlockSpec(memory_space=pltpu.VMEM)] * 3,
        out_specs=pl.BlockSpec(memory_space=pltpu.VMEM),
        scratch_shapes=[
            pltpu.VMEM((N_DEV * m_per, d), jnp.bfloat16),
            pltpu.VMEM((m_per, d), jnp.float32),
            pltpu.VMEM((m_per, d), jnp.bfloat16),
            pltpu.VMEM((m_per, d), jnp.bfloat16),
            pltpu.VMEM((m_per, d), jnp.bfloat16),
            pltpu.VMEM((m_per, d), jnp.bfloat16),
            pltpu.VMEM((m_per, d), jnp.bfloat16),
            pltpu.VMEM((m_per, d), jnp.bfloat16),
            pltpu.SemaphoreType.DMA((2 * (N_DEV - 1),)),
            pltpu.SemaphoreType.DMA((2 * (N_DEV - 1),)),
            pltpu.SemaphoreType.DMA((N_HALF,)),
            pltpu.SemaphoreType.DMA((N_HALF,)),
            pltpu.SemaphoreType.DMA((N_HALF,)),
            pltpu.SemaphoreType.DMA((N_HALF,)),
            pltpu.SemaphoreType.DMA((N_HALF,)),
            pltpu.SemaphoreType.DMA((N_HALF,)),
        ],
        compiler_params=pltpu.CompilerParams(
            collective_id=0, vmem_limit_bytes=60 * 1024 * 1024),
    )
    return f(x.astype(jnp.bfloat16), W1.astype(jnp.bfloat16),
             W2.astype(jnp.bfloat16))


# device time: 136525 ns/iter; 1.9212x vs baseline; 1.0713x over previous
import jax
import jax.numpy as jnp
from jax import lax
from jax.experimental import pallas as pl
from jax.experimental.pallas import tpu as pltpu

N_DEV = 4
F_CHUNK = 2048
N_HALF = 2


def kernel(x, W1, W2):
    m_per, d = x.shape
    f_per = W1.shape[1]
    m_half = m_per // N_HALF

    def compute_partial(xb, w1_ref, w2_ref):
        p = jnp.zeros((m_half, d), jnp.float32)
        for fc in range(f_per // F_CHUNK):
            w1c = w1_ref[:, pl.ds(fc * F_CHUNK, F_CHUNK)]
            hb = jnp.dot(xb, w1c,
                         preferred_element_type=jnp.float32
                         ).astype(jnp.bfloat16)
            hb = hb * jax.nn.sigmoid(hb)
            w2c = w2_ref[pl.ds(fc * F_CHUNK, F_CHUNK), :]
            p = p + jnp.dot(hb, w2c, preferred_element_type=jnp.float32)
        return p

    def body(x_ref, w1_ref, w2_ref, out_ref,
             xfull, pown, pk1, pk2, pk3, fleg, sumb, dirb,
             ag_send, ag_recv,
             k1_send, k2_send, k3_send,
             fleg_recv, sum_recv, dir_recv):
        my = lax.axis_index("i")
        left = lax.rem(my + N_DEV - 1, N_DEV)
        right = lax.rem(my + 1, N_DEV)

        bar = pltpu.get_barrier_semaphore()
        for nbr in (left, right):
            pl.semaphore_signal(bar, inc=1, device_id=(nbr,),
                                device_id_type=pl.DeviceIdType.MESH)
        pl.semaphore_wait(bar, 2)

        def xslice(org, h):
            return pl.ds(org * m_per + h * m_half, m_half)

        def hslice(h):
            return pl.ds(h * m_half, m_half)

        waits = []

        xfull[pl.ds(my * m_per, m_per), :] = x_ref[...].astype(jnp.bfloat16)

        ag_rdmas = {}
        org0 = my
        for h in range(N_HALF):
            rdma = pltpu.make_async_remote_copy(
                src_ref=xfull.at[xslice(org0, h), :],
                dst_ref=xfull.at[xslice(org0, h), :],
                send_sem=ag_send.at[h],
                recv_sem=ag_recv.at[h],
                device_id=(right,),
                device_id_type=pl.DeviceIdType.MESH,
            )
            rdma.start()
            ag_rdmas[(0, h)] = rdma
            waits.append(rdma)
        for h in range(N_HALF):
            pown[hslice(h), :] = compute_partial(
                xfull[xslice(org0, h), :], w1_ref, w2_ref)

        def ag_step(k):
            org = lax.rem(my - k + N_DEV, N_DEV)
            outs = []
            for h in range(N_HALF):
                ag_rdmas[(k - 1, h)].wait_recv()
                if k < N_DEV - 1:
                    rdma = pltpu.make_async_remote_copy(
                        src_ref=xfull.at[xslice(org, h), :],
                        dst_ref=xfull.at[xslice(org, h), :],
                        send_sem=ag_send.at[2 * k + h],
                        recv_sem=ag_recv.at[2 * k + h],
                        device_id=(right,),
                        device_id_type=pl.DeviceIdType.MESH,
                    )
                    rdma.start()
                    ag_rdmas[(k, h)] = rdma
                    waits.append(rdma)
                outs.append(compute_partial(
                    xfull[xslice(org, h), :], w1_ref, w2_ref))
            return outs

        def send(buf, h, sem, target, dst_buf, dst_sem):
            rdma = pltpu.make_async_remote_copy(
                src_ref=buf.at[hslice(h), :],
                dst_ref=dst_buf.at[hslice(h), :],
                send_sem=sem.at[h],
                recv_sem=dst_sem.at[h],
                device_id=(target,),
                device_id_type=pl.DeviceIdType.MESH,
            )
            rdma.start()
            waits.append(rdma)

        def wait_recv(buf, sem, h):
            rdma = pltpu.make_async_remote_copy(
                src_ref=buf.at[hslice(h), :],
                dst_ref=buf.at[hslice(h), :],
                send_sem=sem.at[h],
                recv_sem=sem.at[h],
                device_id=(my,),
                device_id_type=pl.DeviceIdType.MESH,
            )
            rdma.wait_recv()

        for h, p in enumerate(ag_step(1)):
            pk1[hslice(h), :] = p.astype(jnp.bfloat16)

        for h, p in enumerate(ag_step(2)):
            pk2[hslice(h), :] = p.astype(jnp.bfloat16)
            send(pk2, h, k2_send, left, fleg, fleg_recv)

        org3 = lax.rem(my + 1, N_DEV)
        for h in range(N_HALF):
            ag_rdmas[(2, h)].wait_recv()
            pk3[hslice(h), :] = compute_partial(
                xfull[xslice(org3, h), :], w1_ref, w2_ref).astype(jnp.bfloat16)
            send(pk3, h, k3_send, right, dirb, dir_recv)
            wait_recv(fleg, fleg_recv, h)
            pk1[hslice(h), :] = pk1[hslice(h), :] + fleg[hslice(h), :]
            send(pk1, h, k1_send, left, sumb, sum_recv)

        for h in range(N_HALF):
            wait_recv(sumb, sum_recv, h)
            wait_recv(dirb, dir_recv, h)
            out_ref[hslice(h), :] = (
                pown[hslice(h), :]
                + sumb[hslice(h), :].astype(jnp.float32)
                + dirb[hslice(h), :].astype(jnp.float32))

        for rdma in waits:
            rdma.wait_send()

    f = pl.pallas_call(
        body,
        out_shape=jax.ShapeDtypeStruct((m_per, d), jnp.float32),
        in_specs=[pl.BlockSpec(memory_space=pltpu.VMEM)] * 3,
        out_specs=pl.BlockSpec(memory_space=pltpu.VMEM),
        scratch_shapes=[
            pltpu.VMEM((N_DEV * m_per, d), jnp.bfloat16),
            pltpu.VMEM((m_per, d), jnp.float32),
            pltpu.VMEM((m_per, d), jnp.bfloat16),
            pltpu.VMEM((m_per, d), jnp.bfloat16),
            pltpu.VMEM((m_per, d), jnp.bfloat16),
            pltpu.VMEM((m_per, d), jnp.bfloat16),
            pltpu.VMEM((m_per, d), jnp.bfloat16),
            pltpu.VMEM((m_per, d), jnp.bfloat16),
            pltpu.SemaphoreType.DMA((2 * (N_DEV - 1),)),
            pltpu.SemaphoreType.DMA((2 * (N_DEV - 1),)),
            pltpu.SemaphoreType.DMA((N_HALF,)),
            pltpu.SemaphoreType.DMA((N_HALF,)),
            pltpu.SemaphoreType.DMA((N_HALF,)),
            pltpu.SemaphoreType.DMA((N_HALF,)),
            pltpu.SemaphoreType.DMA((N_HALF,)),
            pltpu.SemaphoreType.DMA((N_HALF,)),
        ],
        compiler_params=pltpu.CompilerParams(
            collective_id=0, vmem_limit_bytes=60 * 1024 * 1024),
    )
    return f(x, W1.astype(jnp.bfloat16), W2.astype(jnp.bfloat16))


# device time: 135792 ns/iter; 1.9316x vs baseline; 1.0054x over previous
import jax
import jax.numpy as jnp
from jax import lax
from jax.experimental import pallas as pl
from jax.experimental.pallas import tpu as pltpu

N_DEV = 4
F_CHUNK = 2048
N_HALF = 2


def kernel(x, W1, W2):
    m_per, d = x.shape
    f_per = W1.shape[1]
    m_half = m_per // N_HALF

    def compute_partial(xb, w1_ref, w2_ref):
        p = jnp.zeros((m_half, d), jnp.float32)
        for fc in range(f_per // F_CHUNK):
            w1c = w1_ref[:, pl.ds(fc * F_CHUNK, F_CHUNK)]
            hb = jnp.dot(xb, w1c,
                         preferred_element_type=jnp.float32
                         ).astype(jnp.bfloat16)
            hb = hb * jax.nn.sigmoid(hb)
            w2c = w2_ref[pl.ds(fc * F_CHUNK, F_CHUNK), :]
            p = p + jnp.dot(hb, w2c, preferred_element_type=jnp.float32)
        return p

    def body(x_ref, w1_ref, w2_ref, out_ref,
             xfull, pown, pk1, pk2, pk3, fleg, sumb, dirb,
             ag_send, ag_recv,
             k1_send, k2_send, k3_send,
             fleg_recv, sum_recv, dir_recv):
        my = lax.axis_index("i")
        left = lax.rem(my + N_DEV - 1, N_DEV)
        right = lax.rem(my + 1, N_DEV)

        bar = pltpu.get_barrier_semaphore()
        for nbr in (left, right):
            pl.semaphore_signal(bar, inc=1, device_id=(nbr,),
                                device_id_type=pl.DeviceIdType.MESH)
        pl.semaphore_wait(bar, 2)

        def xslice(org, h):
            return pl.ds(org * m_per + h * m_half, m_half)

        def hslice(h):
            return pl.ds(h * m_half, m_half)

        waits = []

        xfull[pl.ds(my * m_per, m_per), :] = x_ref[...].astype(jnp.bfloat16)

        ag_rdmas = {}
        org0 = my
        for h in range(N_HALF):
            rdma = pltpu.make_async_remote_copy(
                src_ref=xfull.at[xslice(org0, h), :],
                dst_ref=xfull.at[xslice(org0, h), :],
                send_sem=ag_send.at[h],
                recv_sem=ag_recv.at[h],
                device_id=(right,),
                device_id_type=pl.DeviceIdType.MESH,
            )
            rdma.start()
            ag_rdmas[(0, h)] = rdma
            waits.append(rdma)
        pown[hslice(0), :] = compute_partial(
            xfull[xslice(org0, 0), :], w1_ref, w2_ref)

        def ag_step(k):
            org = lax.rem(my - k + N_DEV, N_DEV)
            outs = []
            for h in range(N_HALF):
                ag_rdmas[(k - 1, h)].wait_recv()
                if k < N_DEV - 1:
                    rdma = pltpu.make_async_remote_copy(
                        src_ref=xfull.at[xslice(org, h), :],
                        dst_ref=xfull.at[xslice(org, h), :],
                        send_sem=ag_send.at[2 * k + h],
                        recv_sem=ag_recv.at[2 * k + h],
                        device_id=(right,),
                        device_id_type=pl.DeviceIdType.MESH,
                    )
                    rdma.start()
                    ag_rdmas[(k, h)] = rdma
                    waits.append(rdma)
                outs.append(compute_partial(
                    xfull[xslice(org, h), :], w1_ref, w2_ref))
            return outs

        def send(buf, h, sem, target, dst_buf, dst_sem):
            rdma = pltpu.make_async_remote_copy(
                src_ref=buf.at[hslice(h), :],
                dst_ref=dst_buf.at[hslice(h), :],
                send_sem=sem.at[h],
                recv_sem=dst_sem.at[h],
                device_id=(target,),
                device_id_type=pl.DeviceIdType.MESH,
            )
            rdma.start()
            waits.append(rdma)

        def wait_recv(buf, sem, h):
            rdma = pltpu.make_async_remote_copy(
                src_ref=buf.at[hslice(h), :],
                dst_ref=buf.at[hslice(h), :],
                send_sem=sem.at[h],
                recv_sem=sem.at[h],
                device_id=(my,),
                device_id_type=pl.DeviceIdType.MESH,
            )
            rdma.wait_recv()

        for h, p in enumerate(ag_step(1)):
            pk1[hslice(h), :] = p.astype(jnp.bfloat16)

        for h, p in enumerate(ag_step(2)):
            pk2[hslice(h), :] = p.astype(jnp.bfloat16)
            send(pk2, h, k2_send, left, fleg, fleg_recv)

        org3 = lax.rem(my + 1, N_DEV)
        for h in range(N_HALF):
            ag_rdmas[(2, h)].wait_recv()
            pk3[hslice(h), :] = compute_partial(
                xfull[xslice(org3, h), :], w1_ref, w2_ref).astype(jnp.bfloat16)
            send(pk3, h, k3_send, right, dirb, dir_recv)
            wait_recv(fleg, fleg_recv, h)
            pk1[hslice(h), :] = pk1[hslice(h), :] + fleg[hslice(h), :]
            send(pk1, h, k1_send, left, sumb, sum_recv)

        pown[hslice(1), :] = compute_partial(
            xfull[xslice(org0, 1), :], w1_ref, w2_ref)

        for h in range(N_HALF):
            wait_recv(sumb, sum_recv, h)
            wait_recv(dirb, dir_recv, h)
            out_ref[hslice(h), :] = (
                pown[hslice(h), :]
                + sumb[hslice(h), :].astype(jnp.float32)
                + dirb[hslice(h), :].astype(jnp.float32))

        for rdma in waits:
            rdma.wait_send()

    f = pl.pallas_call(
        body,
        out_shape=jax.ShapeDtypeStruct((m_per, d), jnp.float32),
        in_specs=[pl.BlockSpec(memory_space=pltpu.VMEM)] * 3,
        out_specs=pl.BlockSpec(memory_space=pltpu.VMEM),
        scratch_shapes=[
            pltpu.VMEM((N_DEV * m_per, d), jnp.bfloat16),
            pltpu.VMEM((m_per, d), jnp.float32),
            pltpu.VMEM((m_per, d), jnp.bfloat16),
            pltpu.VMEM((m_per, d), jnp.bfloat16),
            pltpu.VMEM((m_per, d), jnp.bfloat16),
            pltpu.VMEM((m_per, d), jnp.bfloat16),
            pltpu.VMEM((m_per, d), jnp.bfloat16),
            pltpu.VMEM((m_per, d), jnp.bfloat16),
            pltpu.SemaphoreType.DMA((2 * (N_DEV - 1),)),
            pltpu.SemaphoreType.DMA((2 * (N_DEV - 1),)),
            pltpu.SemaphoreType.DMA((N_HALF,)),
            pltpu.SemaphoreType.DMA((N_HALF,)),
            pltpu.SemaphoreType.DMA((N_HALF,)),
            pltpu.SemaphoreType.DMA((N_HALF,)),
            pltpu.SemaphoreType.DMA((N_HALF,)),
            pltpu.SemaphoreType.DMA((N_HALF,)),
        ],
        compiler_params=pltpu.CompilerParams(
            collective_id=0, vmem_limit_bytes=60 * 1024 * 1024),
    )
    return f(x, W1.astype(jnp.bfloat16), W2.astype(jnp.bfloat16))
